# baseline (speedup 1.0000x reference)
import jax
import jax.numpy as jnp
from jax import lax
from jax.experimental import pallas as pl
from jax.experimental.pallas import tpu as pltpu
from jax.experimental.pallas import tpu_sc as plsc

NGRAMS = 20
BATCH = 1024
EMB_DIM = 64
LANES = 16
NW = 32
BPW = BATCH // NW
IDX_PER_W = BPW * NGRAMS
GCHUNK = 128
NCHUNK = IDX_PER_W // GCHUNK
ZROW = IDX_PER_W
NGROUP = BPW // LANES


def _sc_body(idx_hbm, table_hbm, out_hbm, idx_v, rows_v, out_v, sem, gsem):
    wid = lax.axis_index("s") * 2 + lax.axis_index("c")
    base = wid * BPW

    idx_copies = [
        pltpu.make_async_copy(
            idx_hbm.at[pl.ds(s * BATCH + base, BPW)],
            idx_v.at[pl.ds(s * BPW, BPW)],
            sem,
        )
        for s in range(NGRAMS)
    ]
    with jax.named_scope("stage_idx"):
        for c in idx_copies:
            c.start()
        for c in idx_copies:
            c.wait()

    row_copies = [
        pltpu.make_async_copy(
            table_hbm.at[idx_v.at[pl.ds(j * GCHUNK, GCHUNK)]],
            rows_v.at[pl.ds(j * GCHUNK, GCHUNK)],
            gsem,
        )
        for j in range(NCHUNK)
    ]
    for c in row_copies:
        c.start()

    zeros16 = jnp.zeros((LANES,), jnp.float32)
    for dc in range(EMB_DIM // LANES):
        rows_v[ZROW, pl.ds(dc * LANES, LANES)] = zeros16

    lanes = lax.iota(jnp.int32, LANES)

    with jax.named_scope("masks"):
        groups = []
        for g in range(NGROUP):
            v = [
                idx_v[pl.ds(s * BPW + g * LANES, LANES)] for s in range(NGRAMS)
            ]
            pf = [lanes + g * LANES]
            for s in range(1, NGRAMS):
                dup = v[s] == v[0]
                for t in range(1, s):
                    dup = dup | (v[s] == v[t])
                pf.append(jnp.where(dup, ZROW, s * BPW + g * LANES + lanes))
            groups.append(pf)

    with jax.named_scope("gather_wait"):
        for c in row_copies:
            c.wait()

    with jax.named_scope("accum"):
        for g in range(NGROUP):
            pf = groups[g]
            row_out = lanes + g * LANES

            def dbody(d, _):
                col = jnp.full((LANES,), d, jnp.int32)
                acc = plsc.load_gather(rows_v, [pf[0], col])
                for s in range(1, NGRAMS):
                    acc = acc + plsc.load_gather(rows_v, [pf[s], col])
                plsc.store_scatter(out_v, [row_out, col], acc)
                return _

            lax.fori_loop(0, EMB_DIM, dbody, None, unroll=4)

    with jax.named_scope("writeout"):
        pltpu.sync_copy(out_v, out_hbm.at[pl.ds(base, BPW)])


def kernel(input, W):
    idx_lin = jnp.maximum(input.reshape(-1), 0)
    mesh = plsc.VectorSubcoreMesh(core_axis_name="c", subcore_axis_name="s")
    f = pl.kernel(
        _sc_body,
        out_type=jax.ShapeDtypeStruct((BATCH, EMB_DIM), jnp.float32),
        mesh=mesh,
        compiler_params=pltpu.CompilerParams(
            needs_layout_passes=False, use_tc_tiling_on_sc=False
        ),
        scratch_types=[
            pltpu.VMEM((IDX_PER_W,), jnp.int32),
            pltpu.VMEM((IDX_PER_W + 1, EMB_DIM), jnp.float32),
            pltpu.VMEM((BPW, EMB_DIM), jnp.float32),
            pltpu.SemaphoreType.DMA,
            pltpu.SemaphoreType.DMA,
        ],
    )
    return f(idx_lin, W)


# speedup vs baseline: 1.2727x; 1.2727x over previous
import jax
import jax.numpy as jnp
from jax import lax
from jax.experimental import pallas as pl
from jax.experimental.pallas import tpu as pltpu
from jax.experimental.pallas import tpu_sc as plsc

NGRAMS = 20
BATCH = 1024
EMB_DIM = 64
LANES = 16
NW = 32
BPW = BATCH // NW
IDX_PER_W = BPW * NGRAMS
GCHUNK = 128
NCHUNK = IDX_PER_W // GCHUNK
ZROW = IDX_PER_W
NGROUP = BPW // LANES

_TAKE_DNUMS = lax.GatherDimensionNumbers(
    offset_dims=(), collapsed_slice_dims=(0,), start_index_map=(0,)
)


def _take(vec, idx):
    return lax.gather(
        vec,
        idx[:, None],
        _TAKE_DNUMS,
        (1,),
        mode=lax.GatherScatterMode.PROMISE_IN_BOUNDS,
    )


def _sc_body(idx_hbm, table_hbm, out_hbm, idx_v, rows_v, out_v, sem, gsem):
    wid = lax.axis_index("s") * 2 + lax.axis_index("c")
    base = wid * BPW

    idx_copies = [
        pltpu.make_async_copy(
            idx_hbm.at[pl.ds(s * BATCH + base, BPW)],
            idx_v.at[pl.ds(s * BPW, BPW)],
            sem,
        )
        for s in range(NGRAMS)
    ]
    with jax.named_scope("stage_idx"):
        for c in idx_copies:
            c.start()
        for c in idx_copies:
            c.wait()

    row_copies = [
        pltpu.make_async_copy(
            table_hbm.at[idx_v.at[pl.ds(j * GCHUNK, GCHUNK)]],
            rows_v.at[pl.ds(j * GCHUNK, GCHUNK)],
            gsem,
        )
        for j in range(NCHUNK)
    ]
    for c in row_copies:
        c.start()

    zeros16 = jnp.zeros((LANES,), jnp.float32)
    for dc in range(EMB_DIM // LANES):
        rows_v[ZROW, pl.ds(dc * LANES, LANES)] = zeros16

    lanes = lax.iota(jnp.int32, LANES)

    with jax.named_scope("masks"):
        groups = []
        for g in range(NGROUP):
            v = [
                idx_v[pl.ds(s * BPW + g * LANES, LANES)] for s in range(NGRAMS)
            ]
            pf = [lanes + g * LANES]
            for s in range(1, NGRAMS):
                dup = v[s] == v[0]
                for t in range(1, s):
                    dup = dup | (v[s] == v[t])
                pf.append(jnp.where(dup, ZROW, s * BPW + g * LANES + lanes))
            groups.append(pf)

    with jax.named_scope("gather_wait"):
        for c in row_copies:
            c.wait()

    coffs = [dc * LANES + lanes for dc in range(EMB_DIM // LANES)]
    with jax.named_scope("accum"):
        for g in range(NGROUP):
            pf = groups[g]

            def bbody(b, _):
                bsp = jnp.full((LANES,), b, jnp.int32)
                accs = None
                for s in range(NGRAMS):
                    rsp = _take(pf[s], bsp)
                    vals = [
                        plsc.load_gather(rows_v, [rsp, coff]) for coff in coffs
                    ]
                    if accs is None:
                        accs = vals
                    else:
                        accs = [a + v for a, v in zip(accs, vals)]
                for dc in range(EMB_DIM // LANES):
                    out_v[b + g * LANES, pl.ds(dc * LANES, LANES)] = accs[dc]
                return _

            lax.fori_loop(0, LANES, bbody, None)

    with jax.named_scope("writeout"):
        pltpu.sync_copy(out_v, out_hbm.at[pl.ds(base, BPW)])


def kernel(input, W):
    idx_lin = jnp.maximum(input.reshape(-1), 0)
    mesh = plsc.VectorSubcoreMesh(core_axis_name="c", subcore_axis_name="s")
    f = pl.kernel(
        _sc_body,
        out_type=jax.ShapeDtypeStruct((BATCH, EMB_DIM), jnp.float32),
        mesh=mesh,
        compiler_params=pltpu.CompilerParams(
            needs_layout_passes=False, use_tc_tiling_on_sc=False
        ),
        scratch_types=[
            pltpu.VMEM((IDX_PER_W,), jnp.int32),
            pltpu.VMEM((IDX_PER_W + 1, EMB_DIM), jnp.float32),
            pltpu.VMEM((BPW, EMB_DIM), jnp.float32),
            pltpu.SemaphoreType.DMA,
            pltpu.SemaphoreType.DMA,
        ],
    )
    return f(idx_lin, W)
